# baseline (speedup 1.0000x reference)
import functools

import jax
import jax.numpy as jnp
from jax.experimental import pallas as pl
from jax.experimental.pallas import tpu as pltpu
from jax.experimental.pallas import tpu_sc as plsc

_N = 2
_NNZ = 2


def _scs_body(x_h, rows_h, cols_h, vals_h, out_h,
              x_s, rows_s, cols_s, vals_s, out_s, sem):
    c0 = pltpu.make_async_copy(x_h, x_s, sem)
    c1 = pltpu.make_async_copy(rows_h, rows_s, sem)
    c2 = pltpu.make_async_copy(cols_h, cols_s, sem)
    c3 = pltpu.make_async_copy(vals_h, vals_s, sem)
    c0.start()
    c1.start()
    c2.start()
    c3.start()
    c0.wait()
    c1.wait()
    c2.wait()
    c3.wait()
    for j in range(_N):
        out_s[j] = 0.0
    for i in range(_NNZ):
        r = rows_s[i]
        c = cols_s[i]
        out_s[r] = out_s[r] + vals_s[i] * x_s[c]
    pltpu.sync_copy(out_s, out_h)


_scs_call = functools.partial(
    pl.kernel,
    out_type=jax.ShapeDtypeStruct((_N,), jnp.float32),
    mesh=plsc.ScalarSubcoreMesh(axis_name="c", num_cores=1),
    scratch_types=[
        pltpu.SMEM((_N,), jnp.float32),
        pltpu.SMEM((_NNZ,), jnp.int32),
        pltpu.SMEM((_NNZ,), jnp.int32),
        pltpu.SMEM((_NNZ,), jnp.float32),
        pltpu.SMEM((_N,), jnp.float32),
        pltpu.SemaphoreType.DMA,
    ],
    compiler_params=pltpu.CompilerParams(needs_layout_passes=False,
                                         skip_device_barrier=True),
)(_scs_body)


@jax.jit
def kernel(x, rows, cols, vals):
    return _scs_call(x, rows, cols, vals)


# speedup vs baseline: 1.0046x; 1.0046x over previous
import functools

import jax
import jax.numpy as jnp
from jax.experimental import pallas as pl
from jax.experimental.pallas import tpu as pltpu
from jax.experimental.pallas import tpu_sc as plsc

_N = 2
_NNZ = 2


def _scs_body(x_h, rows_h, cols_h, vals_h, out_h,
              x_s, rows_s, cols_s, vals_s, out_s, sem):
    c0 = pltpu.make_async_copy(x_h, x_s, sem)
    c1 = pltpu.make_async_copy(rows_h, rows_s, sem)
    c2 = pltpu.make_async_copy(cols_h, cols_s, sem)
    c3 = pltpu.make_async_copy(vals_h, vals_s, sem)
    c0.start()
    c1.start()
    c2.start()
    c3.start()
    c0.wait()
    c1.wait()
    c2.wait()
    c3.wait()
    for j in range(_N):
        out_s[j] = 0.0
    for i in range(_NNZ):
        r = rows_s[i]
        c = cols_s[i]
        out_s[r] = out_s[r] + vals_s[i] * x_s[c]
    pltpu.sync_copy(out_s, out_h)


_scs_call = functools.partial(
    pl.kernel,
    out_type=jax.ShapeDtypeStruct((_N,), jnp.float32),
    mesh=plsc.ScalarSubcoreMesh(axis_name="c", num_cores=1),
    scratch_types=[
        pltpu.SMEM((_N,), jnp.float32),
        pltpu.SMEM((_NNZ,), jnp.int32),
        pltpu.SMEM((_NNZ,), jnp.int32),
        pltpu.SMEM((_NNZ,), jnp.float32),
        pltpu.SMEM((_N,), jnp.float32),
        pltpu.SemaphoreType.DMA,
    ],
    compiler_params=pltpu.CompilerParams(needs_layout_passes=False),
)(_scs_body)


@jax.jit
def kernel(x, rows, cols, vals):
    return _scs_call(x, rows, cols, vals)
